# baseline (speedup 1.0000x reference)
import functools

import jax
import jax.numpy as jnp
from jax import lax
from jax.experimental import pallas as pl
from jax.experimental.pallas import tpu as pltpu
from jax.experimental.pallas import tpu_sc as plsc

LANES = 16
RING = 8


@functools.lru_cache(maxsize=None)
def _build(B, D, S, nf, ns, num_cores, num_subcores):
    nw = num_cores * num_subcores
    assert B % (8 * nw) == 0
    assert S <= 1 << 17 and nf * ns <= 1 << 10
    bpw = B // nw
    groups = bpw // LANES
    waves = bpw // LANES

    mesh = plsc.VectorSubcoreMesh(core_axis_name="c", subcore_axis_name="s")

    @functools.partial(
        pl.kernel,
        mesh=mesh,
        out_type=jax.ShapeDtypeStruct((B,), jnp.float32),
        compiler_params=pltpu.CompilerParams(needs_layout_passes=False),
        scratch_types=[
            pltpu.VMEM((bpw,), jnp.int32),
            pltpu.VMEM((bpw,), jnp.int32),
            pltpu.VMEM((bpw,), jnp.int32),
            pltpu.VMEM((bpw,), jnp.int32),
            pltpu.VMEM((bpw,), jnp.float32),
            pltpu.VMEM((LANES, D), jnp.float32),
            pltpu.VMEM((LANES, D), jnp.float32),
        ] + [pltpu.VMEM((D, 128), jnp.float32) for _ in range(RING)]
          + [pltpu.SemaphoreType.DMA for _ in range(RING)]
          + [pltpu.SemaphoreType.DMA, pltpu.SemaphoreType.DMA],
    )
    def emos(wt_hbm, ft_hbm, sid_hbm, fid_hbm, stp_hbm, out_hbm,
             sid_v, fid_v, stp_v, key_v,
             out_v, ftA, ftB, *ring_and_sems):
        blks = ring_and_sems[:RING]
        dsems = ring_and_sems[RING:2 * RING]
        sem_fA, sem_fB = ring_and_sems[2 * RING:]

        wid = lax.axis_index("s") * num_cores + lax.axis_index("c")
        base = wid * bpw

        def fire_feat(kbase, buf, sem):
            pltpu.async_copy(
                ft_hbm.at[pl.ds(base + kbase, LANES), :], buf, sem)

        def drain_feat(buf, sem):
            pltpu.make_async_copy(
                ft_hbm.at[pl.ds(0, LANES), :], buf, sem).wait()

        fire_feat(0, ftA, sem_fA)
        pltpu.sync_copy(sid_hbm.at[pl.ds(base, bpw)], sid_v)
        pltpu.sync_copy(fid_hbm.at[pl.ds(base, bpw)], fid_v)
        pltpu.sync_copy(stp_hbm.at[pl.ds(base, bpw)], stp_v)

        def idx_body(g, carry):
            sl = pl.ds(g * LANES, LANES)
            combo = (fid_v[sl] % nf) * ns + (stp_v[sl] % ns)
            key_v[sl] = (combo << 17) | sid_v[sl]
            return carry

        lax.fori_loop(0, groups, idx_body, 0)

        def fire(key, slot):
            s = key & 131071
            q0 = pl.multiple_of((key >> 17) * D, D)
            u0 = pl.multiple_of((s >> 7) * 128, 128)
            pltpu.async_copy(
                wt_hbm.at[pl.ds(q0, D), pl.ds(u0, 128)], blks[slot],
                dsems[slot])

        key0 = key_v[pl.ds(0, LANES)]
        for r in range(RING):
            fire(key0[r], r)
        fire_feat(LANES, ftB, sem_fB)

        iota = lax.iota(jnp.int32, LANES)

        def one_wave(k, buf, sem):
            kbase = k * LANES
            keys = key_v[pl.ds(kbase, LANES)]
            nxt = lax.min(kbase + LANES, bpw - LANES)
            keysn = key_v[pl.ds(nxt, LANES)]
            drain_feat(buf, sem)
            acc = jnp.zeros((LANES,), jnp.float32)
            for h in range(2):
                for r in range(RING):
                    lane = h * RING + r
                    pltpu.make_async_copy(
                        wt_hbm.at[pl.ds(0, D), pl.ds(0, 128)], blks[r],
                        dsems[r]).wait()
                    key = keys[lane]
                    l = jnp.full((LANES,), key & 127, jnp.int32)
                    p = jnp.zeros((LANES,), jnp.float32)
                    for kk in range(D // LANES):
                        w = plsc.load_gather(blks[r], [kk * LANES + iota, l])
                        f = buf[lane, pl.ds(kk * LANES, LANES)]
                        p = p + w * f
                    acc = jnp.where(iota == lane, jnp.sum(p), acc)
                    if h == 0:
                        fire(keys[lane + RING], r)
                    else:
                        @pl.when(k < waves - 1)
                        def _():
                            fire(keysn[lane - RING], r)

            out_v[pl.ds(kbase, LANES)] = acc

            @pl.when(k + 2 < waves)
            def _():
                fire_feat(kbase + 2 * LANES, buf, sem)

        def pair_body(kk, carry):
            one_wave(2 * kk, ftA, sem_fA)
            one_wave(2 * kk + 1, ftB, sem_fB)
            return carry

        lax.fori_loop(0, waves // 2, pair_body, 0)
        pltpu.sync_copy(out_v, out_hbm.at[pl.ds(base, bpw)])

    return emos


@functools.lru_cache(maxsize=None)
def _build_bias(B, S, nf, ns, num_cores, num_subcores):
    nw = num_cores * num_subcores
    bpw = B // nw
    groups = bpw // LANES

    mesh = plsc.VectorSubcoreMesh(core_axis_name="c", subcore_axis_name="s")

    @functools.partial(
        pl.kernel,
        mesh=mesh,
        out_type=jax.ShapeDtypeStruct((B,), jnp.float32),
        compiler_params=pltpu.CompilerParams(needs_layout_passes=False),
        scratch_types=[
            pltpu.VMEM((bpw,), jnp.int32),
            pltpu.VMEM((bpw,), jnp.int32),
            pltpu.VMEM((bpw,), jnp.int32),
            pltpu.VMEM((bpw,), jnp.float32),
            pltpu.VMEM((bpw,), jnp.float32),
            pltpu.SemaphoreType.DMA,
            pltpu.SemaphoreType.DMA,
        ],
    )
    def bias_add(dots_hbm, btf_hbm, sid_hbm, fid_hbm, stp_hbm, out_hbm,
                 fid_v, stp_v, bidx_v, bias_v, dot_v, sem_b, sem_d):
        wid = lax.axis_index("s") * num_cores + lax.axis_index("c")
        base = wid * bpw

        d_cp = pltpu.async_copy(dots_hbm.at[pl.ds(base, bpw)], dot_v, sem_d)
        pltpu.sync_copy(sid_hbm.at[pl.ds(base, bpw)], bidx_v)
        pltpu.sync_copy(fid_hbm.at[pl.ds(base, bpw)], fid_v)
        pltpu.sync_copy(stp_hbm.at[pl.ds(base, bpw)], stp_v)

        def idx_body(g, carry):
            sl = pl.ds(g * LANES, LANES)
            combo = (fid_v[sl] % nf) * ns + (stp_v[sl] % ns)
            bidx_v[sl] = combo * S + bidx_v[sl]
            return carry

        lax.fori_loop(0, groups, idx_body, 0)
        pltpu.async_copy(btf_hbm.at[bidx_v], bias_v, sem_b).wait()
        d_cp.wait()

        def add_body(g, carry):
            sl = pl.ds(g * LANES, LANES)
            dot_v[sl] = dot_v[sl] + bias_v[sl]
            return carry

        lax.fori_loop(0, groups, add_body, 0)
        pltpu.sync_copy(dot_v, out_hbm.at[pl.ds(base, bpw)])

    return bias_add


def kernel(features, station_id, forecast_id, step_id, weights, biases):
    B, D = features.shape
    S, nf, ns, _ = weights.shape
    wt = weights.transpose(1, 2, 3, 0).reshape(nf * ns * D, S)
    btf = biases.transpose(1, 2, 3, 0).reshape(nf * ns * S)
    info = plsc.get_sparse_core_info()
    fn = _build(B, D, S, nf, ns, info.num_cores, info.num_subcores)
    fn2 = _build_bias(B, S, nf, ns, info.num_cores, info.num_subcores)
    sid = station_id.astype(jnp.int32)
    fid = forecast_id.astype(jnp.int32)
    stp = step_id.astype(jnp.int32)
    dots = fn(wt, features, sid, fid, stp)
    return fn2(dots, btf, sid, fid, stp)
